# baseline (speedup 1.0000x reference)
import jax
import jax.numpy as jnp
from jax import lax
from jax.experimental import pallas as pl
from jax.experimental.pallas import tpu as pltpu

MAXP = 2048
SEQ = 2048
D = 64
T = 2 * MAXP + 1
SCALE = D ** (-0.5)
PAD = 4352
NB = 128
M = SEQ // NB
NSLOT = 4


def _tc_body(tab_ref, o_ref, st, sem):
    b = pl.program_id(0)
    slot = lax.rem(b, NSLOT)
    shift = lax.rem(PAD - b, PAD)
    rolled = pltpu.roll(tab_ref[...], shift, axis=1) * SCALE

    def drain(s):
        pltpu.make_async_copy(
            o_ref.at[pl.ds(0, M)], o_ref.at[pl.ds(0, M)], s
        ).wait()

    @pl.when(b >= NSLOT)
    def _():
        drain(sem.at[slot])

    st[slot] = rolled

    is0 = jnp.where(b == 0, 1, 0)
    for m in range(M):
        a = m + is0
        i = MAXP - (128 * a + b)
        pltpu.make_async_copy(
            st.at[slot, :, pl.ds(a * 128, SEQ)], o_ref.at[i], sem.at[slot]
        ).start()

    @pl.when(b == NB - 1)
    def _():
        for d in range(NSLOT):
            drain(sem.at[lax.rem(slot + 1 + d, NSLOT)])


def _materialize(tab_pad):
    return pl.pallas_call(
        _tc_body,
        grid=(NB,),
        in_specs=[pl.BlockSpec((D, PAD), lambda b: (0, 0))],
        out_specs=pl.BlockSpec(memory_space=pl.ANY),
        out_shape=jax.ShapeDtypeStruct((SEQ, D, SEQ), jnp.float32),
        scratch_shapes=[
            pltpu.VMEM((NSLOT, D, PAD), jnp.float32),
            pltpu.SemaphoreType.DMA((NSLOT,)),
        ],
    )(tab_pad)


@jax.jit
def kernel(x, rel_pos_emb):
    del x
    tab_pad = jnp.pad(rel_pos_emb.T, ((0, 0), (0, PAD - T)))
    out_phys = _materialize(tab_pad)
    return out_phys.transpose(0, 2, 1)


# speedup vs baseline: 1.0114x; 1.0114x over previous
import jax
import jax.numpy as jnp
from jax import lax
from jax.experimental import pallas as pl
from jax.experimental.pallas import tpu as pltpu

MAXP = 2048
SEQ = 2048
D = 64
T = 2 * MAXP + 1
SCALE = D ** (-0.5)
PAD = 4352
NB = 128
M = SEQ // NB
NSLOT = 2


def _tc_body(tab_ref, o_ref, st, sem):
    b = pl.program_id(0)
    slot = lax.rem(b, NSLOT)
    shift = lax.rem(PAD - b, PAD)
    rolled = pltpu.roll(tab_ref[...], shift, axis=1) * SCALE

    def drain(s):
        pltpu.make_async_copy(
            o_ref.at[pl.ds(0, M)], o_ref.at[pl.ds(0, M)], s
        ).wait()

    @pl.when(b >= NSLOT)
    def _():
        drain(sem.at[slot])

    st[slot] = rolled

    is0 = jnp.where(b == 0, 1, 0)
    for m in range(M):
        a = m + is0
        i = MAXP - (128 * a + b)
        pltpu.make_async_copy(
            st.at[slot, :, pl.ds(a * 128, SEQ)], o_ref.at[i], sem.at[slot]
        ).start()

    @pl.when(b == NB - 1)
    def _():
        for d in range(NSLOT):
            drain(sem.at[lax.rem(slot + 1 + d, NSLOT)])


def _materialize(tab_pad):
    return pl.pallas_call(
        _tc_body,
        grid=(NB,),
        in_specs=[pl.BlockSpec((D, PAD), lambda b: (0, 0))],
        out_specs=pl.BlockSpec(memory_space=pl.ANY),
        out_shape=jax.ShapeDtypeStruct((SEQ, D, SEQ), jnp.float32),
        scratch_shapes=[
            pltpu.VMEM((NSLOT, D, PAD), jnp.float32),
            pltpu.SemaphoreType.DMA((NSLOT,)),
        ],
    )(tab_pad)


@jax.jit
def kernel(x, rel_pos_emb):
    del x
    tab_pad = jnp.pad(rel_pos_emb.T, ((0, 0), (0, PAD - T)))
    out_phys = _materialize(tab_pad)
    return out_phys.transpose(0, 2, 1)
